# baseline (speedup 1.0000x reference)
import functools

import jax
import jax.numpy as jnp
from jax import lax
from jax.experimental import pallas as pl
from jax.experimental.pallas import tpu as pltpu
from jax.experimental.pallas import tpu_sc as plsc

_N_EXPERTS = 64
_D = 64
_B = 16384

_NC = 2
_NS = 16
_NW = _NC * _NS
_BPW = _B // _NW


def _rsqrt_newton(s):
    i = lax.bitcast_convert_type(s, jnp.int32)
    y = lax.bitcast_convert_type(jnp.int32(0x5F3759DF) - (i >> 1), jnp.float32)
    for _ in range(4):
        y = y * (1.5 - 0.5 * s * y * y)
    return y


_mesh = plsc.VectorSubcoreMesh(
    core_axis_name="c", subcore_axis_name="s", num_cores=_NC, num_subcores=_NS
)


_KERNEL_KWARGS = dict(
    mesh=_mesh,
    out_type=jax.ShapeDtypeStruct((_B, _D), jnp.float32),
    scratch_types=[
        pltpu.VMEM((_N_EXPERTS, _D), jnp.float32),
        pltpu.VMEM((_BPW,), jnp.int32),
        pltpu.VMEM((_BPW, _D), jnp.float32),
        pltpu.HBM((_NW * _N_EXPERTS, _D), jnp.float32),
        pltpu.SemaphoreType.DMA,
    ],
    compiler_params=pltpu.CompilerParams(use_tc_tiling_on_sc=False, needs_layout_passes=False),
)


def _lookup_body(tab_hbm, idx_hbm, out_hbm, tab_v, idx_v, rows_v, tab_sh, sem):
    sid = lax.axis_index("s")
    wid = sid * _NC + lax.axis_index("c")
    base = wid * _BPW
    pltpu.sync_copy(idx_hbm.at[pl.ds(base, _BPW)], idx_v)
    pltpu.sync_copy(tab_hbm, tab_v)

    lanes = lax.iota(jnp.int32, 16)
    for g in range(_N_EXPERTS // 16):
        rows16 = lanes + g * 16
        acc = jnp.zeros((16,), jnp.float32)
        for d in range(_D):
            dfull = jnp.full((16,), d, jnp.int32)
            v = plsc.load_gather(tab_v, [rows16, dfull])
            acc = acc + v * v
        scale = jnp.where(acc > 0.0, _rsqrt_newton(acc), 0.0)
        for d in range(_D):
            dfull = jnp.full((16,), d, jnp.int32)
            v = plsc.load_gather(tab_v, [rows16, dfull])
            plsc.store_scatter(tab_v, [rows16, dfull], v * scale)

    pltpu.sync_copy(tab_v, tab_sh.at[pl.ds(wid * _N_EXPERTS, _N_EXPERTS)])
    for g in range(_BPW // 16):
        idx_v[pl.ds(g * 16, 16)] = idx_v[pl.ds(g * 16, 16)] + wid * _N_EXPERTS
    pltpu.async_copy(tab_sh.at[idx_v], rows_v, sem).wait()
    pltpu.sync_copy(rows_v, out_hbm.at[pl.ds(base, _BPW)])


_lookup_kernel = pl.kernel(_lookup_body, **_KERNEL_KWARGS)


def kernel(experts, table):
    return _lookup_kernel(table, experts.astype(jnp.int32))
